# Initial kernel scaffold; baseline (speedup 1.0000x reference)
import jax
import jax.numpy as jnp
from jax.experimental import pallas as pl


def kernel(z, z1, z2, z3, emb_table, W1, b1, W2, b2, W3, b3):
    raise NotImplementedError("write your pallas kernel here")


# speedup vs baseline: 1.0339x; 1.0339x over previous
import functools

import jax
import jax.numpy as jnp
from jax import lax
from jax.experimental import pallas as pl
from jax.experimental.pallas import tpu as pltpu
from jax.experimental.pallas import tpu_sc as plsc

NC = 2
NS = 16
L = 16
NW = NC * NS
B = 16384
D = 128
D1 = 50
BPW = B // NW
S = 256
NSUB = BPW // S
GI = 128
NG = S // GI

_mesh = plsc.VectorSubcoreMesh(core_axis_name="c", subcore_axis_name="s")


@functools.partial(
    pl.kernel,
    out_type=(
        jax.ShapeDtypeStruct((B, D), jnp.float32),
        jax.ShapeDtypeStruct((B * D1,), jnp.float32),
        jax.ShapeDtypeStruct((B, D), jnp.float32),
        jax.ShapeDtypeStruct((B, D), jnp.float32),
    ),
    mesh=_mesh,
    compiler_params=pltpu.CompilerParams(needs_layout_passes=False),
    scratch_types=[
        pltpu.VMEM((NG, GI), jnp.int32),
        pltpu.VMEM((S,), jnp.float32),
        pltpu.VMEM((S,), jnp.float32),
        pltpu.VMEM((S,), jnp.float32),
        pltpu.VMEM((S, D), jnp.float32),
        pltpu.VMEM((S * D1,), jnp.float32),
        pltpu.VMEM((S, D), jnp.float32),
        pltpu.VMEM((S, D), jnp.float32),
        pltpu.VMEM((64,), jnp.float32),
        pltpu.VMEM((64,), jnp.float32),
        pltpu.VMEM((D,), jnp.float32),
        pltpu.VMEM((D,), jnp.float32),
        pltpu.VMEM((D,), jnp.float32),
        pltpu.VMEM((D,), jnp.float32),
        pltpu.SemaphoreType.DMA,
        pltpu.SemaphoreType.DMA,
    ],
)
def _sc_kernel(z2d_hbm, z1_hbm, z2_hbm, z3_hbm, tab_hbm,
               w1_hbm, b1_hbm, w2_hbm, b2_hbm, w3_hbm, b3_hbm,
               emb_hbm, o1_hbm, o2_hbm, o3_hbm,
               idx_v, z1c_v, z2c_v, z3c_v, rows_v, o1_v, o2_v, o3_v,
               w1_v, b1_v, w2_v, b2_v, w3_v, b3_v, gsem, osem):
    wid = lax.axis_index("s") * NC + lax.axis_index("c")
    base = wid * BPW

    pltpu.sync_copy(w1_hbm, w1_v)
    pltpu.sync_copy(b1_hbm, b1_v)
    pltpu.sync_copy(w2_hbm, w2_v)
    pltpu.sync_copy(b2_hbm, b2_v)
    pltpu.sync_copy(w3_hbm, w3_v)
    pltpu.sync_copy(b3_hbm, b3_v)

    iota = lax.iota(jnp.int32, L)
    offs1 = (0, 16, 32, 34)
    offsd = tuple(range(0, D, L))

    def hoist(wref, bref, offs):
        return ([plsc.load_gather(wref, [iota + o]) for o in offs],
                [plsc.load_gather(bref, [iota + o]) for o in offs])

    for sub in range(NSUB):
        sb = base + sub * S
        rb = wid * (BPW // GI) + sub * NG

        pltpu.sync_copy(z2d_hbm.at[pl.ds(rb, NG)], idx_v)
        gd = [
            pltpu.async_copy(tab_hbm.at[idx_v.at[j]],
                             rows_v.at[pl.ds(j * GI, GI)], gsem)
            for j in range(NG)
        ]
        pltpu.sync_copy(z1_hbm.at[pl.ds(sb, S)], z1c_v)
        pltpu.sync_copy(z2_hbm.at[pl.ds(sb, S)], z2c_v)
        pltpu.sync_copy(z3_hbm.at[pl.ds(sb, S)], z3c_v)

        for zc, buf, wref, bref, offs, flat_w in (
            (z1c_v, o1_v, w1_v, b1_v, offs1, D1),
            (z2c_v, o2_v, w2_v, b2_v, offsd, 0),
            (z3c_v, o3_v, w3_v, b3_v, offsd, 0),
        ):
            wch, bch = hoist(wref, bref, offs)
            cols = [iota + o for o in offs]

            def body(i, c, zc=zc, buf=buf, wch=wch, bch=bch, cols=cols,
                     flat_w=flat_w):
                ibc = jnp.full((L,), i, jnp.int32)
                zb = plsc.load_gather(zc, [ibc])
                if flat_w:
                    rowb = jnp.full((L,), i * flat_w, jnp.int32)
                    for k in range(len(cols)):
                        plsc.store_scatter(buf, [rowb + cols[k]],
                                           zb * wch[k] + bch[k])
                else:
                    for k in range(len(cols)):
                        plsc.store_scatter(buf, [ibc, cols[k]],
                                           zb * wch[k] + bch[k])
                return c

            lax.fori_loop(0, S, body, 0)

        for g in gd:
            g.wait()

        od = [
            pltpu.async_copy(rows_v, emb_hbm.at[pl.ds(sb, S)], osem),
            pltpu.async_copy(o1_v, o1_hbm.at[pl.ds(sb * D1, S * D1)], osem),
            pltpu.async_copy(o2_v, o2_hbm.at[pl.ds(sb, S)], osem),
            pltpu.async_copy(o3_v, o3_hbm.at[pl.ds(sb, S)], osem),
        ]
        for o in od:
            o.wait()


def kernel(z, z1, z2, z3, emb_table, W1, b1, W2, b2, W3, b3):
    z2d = z.astype(jnp.int32).reshape(B // GI, GI)
    w1 = jnp.pad(W1.reshape(-1), (0, 64 - D1))
    b1p = jnp.pad(b1, (0, 64 - D1))
    w2 = W2.reshape(-1)
    w3 = W3.reshape(-1)
    emb, o1, o2, o3 = _sc_kernel(z2d, z1, z2, z3, emb_table,
                                 w1, b1p, w2, b2, w3, b3)
    return emb, o1.reshape(B, D1), o2, o3


# speedup vs baseline: 1.1991x; 1.1598x over previous
import functools

import jax
import jax.numpy as jnp
from jax import lax
from jax.experimental import pallas as pl
from jax.experimental.pallas import tpu as pltpu
from jax.experimental.pallas import tpu_sc as plsc

NC = 2
NS = 16
L = 16
NW = NC * NS
B = 16384
D = 128
D1 = 50
BPW = B // NW
S = 128
NSUB = BPW // S

_mesh = plsc.VectorSubcoreMesh(core_axis_name="c", subcore_axis_name="s")


@functools.partial(
    pl.kernel,
    out_type=(
        jax.ShapeDtypeStruct((B, D), jnp.float32),
        jax.ShapeDtypeStruct((B, D1), jnp.float32),
        jax.ShapeDtypeStruct((B, D), jnp.float32),
        jax.ShapeDtypeStruct((B, D), jnp.float32),
    ),
    mesh=_mesh,
    compiler_params=pltpu.CompilerParams(needs_layout_passes=False),
    scratch_types=[
        pltpu.VMEM((S,), jnp.int32),
        pltpu.VMEM((S,), jnp.float32),
        pltpu.VMEM((S,), jnp.float32),
        pltpu.VMEM((S,), jnp.float32),
        pltpu.VMEM((S, D), jnp.float32),
        pltpu.VMEM((S, D1), jnp.float32),
        pltpu.VMEM((S, D), jnp.float32),
        pltpu.VMEM((S, D), jnp.float32),
        pltpu.VMEM((D1,), jnp.float32),
        pltpu.VMEM((D1,), jnp.float32),
        pltpu.VMEM((D,), jnp.float32),
        pltpu.VMEM((D,), jnp.float32),
        pltpu.VMEM((D,), jnp.float32),
        pltpu.VMEM((D,), jnp.float32),
        pltpu.SemaphoreType.DMA,
        pltpu.SemaphoreType.DMA,
    ],
)
def _sc_kernel(z_hbm, z1_hbm, z2_hbm, z3_hbm, tab_hbm,
               w1_hbm, b1_hbm, w2_hbm, b2_hbm, w3_hbm, b3_hbm,
               emb_hbm, o1_hbm, o2_hbm, o3_hbm,
               idx_v, z1c_v, z2c_v, z3c_v, rows_v, o1_v, o2_v, o3_v,
               w1_v, b1_v, w2_v, b2_v, w3_v, b3_v, gsem, osem):
    wid = lax.axis_index("s") * NC + lax.axis_index("c")
    base = wid * BPW

    pltpu.sync_copy(w1_hbm, w1_v)
    pltpu.sync_copy(b1_hbm, b1_v)
    pltpu.sync_copy(w2_hbm, w2_v)
    pltpu.sync_copy(b2_hbm, b2_v)
    pltpu.sync_copy(w3_hbm, w3_v)
    pltpu.sync_copy(b3_hbm, b3_v)

    iota = lax.iota(jnp.int32, L)
    offs1 = (0, 16, 32, 34)
    offsd = tuple(range(0, D, L))

    def hoist(wref, bref, offs):
        return ([plsc.load_gather(wref, [iota + o]) for o in offs],
                [plsc.load_gather(bref, [iota + o]) for o in offs])

    for sub in range(NSUB):
        sb = base + sub * S

        pltpu.sync_copy(z_hbm.at[pl.ds(sb, S)], idx_v)
        gd = [pltpu.async_copy(tab_hbm.at[idx_v], rows_v, gsem)]
        pltpu.sync_copy(z1_hbm.at[pl.ds(sb, S)], z1c_v)
        pltpu.sync_copy(z2_hbm.at[pl.ds(sb, S)], z2c_v)
        pltpu.sync_copy(z3_hbm.at[pl.ds(sb, S)], z3c_v)

        for zc, buf, wref, bref, offs in (
            (z1c_v, o1_v, w1_v, b1_v, offs1),
            (z2c_v, o2_v, w2_v, b2_v, offsd),
            (z3c_v, o3_v, w3_v, b3_v, offsd),
        ):
            wch, bch = hoist(wref, bref, offs)
            cols = [iota + o for o in offs]

            def body(i, c, zc=zc, buf=buf, wch=wch, bch=bch, cols=cols):
                ibc = jnp.full((L,), i, jnp.int32)
                zb = plsc.load_gather(zc, [ibc])
                for k in range(len(cols)):
                    plsc.store_scatter(buf, [ibc, cols[k]],
                                       zb * wch[k] + bch[k])
                return c

            lax.fori_loop(0, S, body, 0)

        for g in gd:
            g.wait()

        od = [
            pltpu.async_copy(rows_v, emb_hbm.at[pl.ds(sb, S)], osem),
            pltpu.async_copy(o1_v, o1_hbm.at[pl.ds(sb, S)], osem),
            pltpu.async_copy(o2_v, o2_hbm.at[pl.ds(sb, S)], osem),
            pltpu.async_copy(o3_v, o3_hbm.at[pl.ds(sb, S)], osem),
        ]
        for o in od:
            o.wait()


def kernel(z, z1, z2, z3, emb_table, W1, b1, W2, b2, W3, b3):
    w1 = W1.reshape(-1)
    w2 = W2.reshape(-1)
    w3 = W3.reshape(-1)
    return _sc_kernel(z.astype(jnp.int32), z1, z2, z3, emb_table,
                      w1, b1, w2, b2, w3, b3)


# speedup vs baseline: 1.5926x; 1.3281x over previous
import functools

import jax
import jax.numpy as jnp
from jax import lax
from jax.experimental import pallas as pl
from jax.experimental.pallas import tpu as pltpu
from jax.experimental.pallas import tpu_sc as plsc

NC = 2
NS = 16
L = 16
NW = NC * NS
B = 16384
D = 128
D1 = 50
BPW = B // NW
GI = 128
NG = BPW // GI

_mesh = plsc.VectorSubcoreMesh(core_axis_name="c", subcore_axis_name="s")


@functools.partial(
    pl.kernel,
    out_type=jax.ShapeDtypeStruct((B, D), jnp.float32),
    mesh=_mesh,
    compiler_params=pltpu.CompilerParams(needs_layout_passes=False),
    scratch_types=[
        pltpu.VMEM((NG, GI), jnp.int32),
        pltpu.VMEM((GI, D), jnp.float32),
        pltpu.VMEM((GI, D), jnp.float32),
        pltpu.SemaphoreType.DMA,
        pltpu.SemaphoreType.DMA,
    ],
)
def _sc_emb(z2d_hbm, tab_hbm, emb_hbm, idx_v, rows0_v, rows1_v, gsem, osem):
    wid = lax.axis_index("s") * NC + lax.axis_index("c")
    base = wid * BPW
    bufs = (rows0_v, rows1_v)

    pltpu.sync_copy(z2d_hbm.at[pl.ds(wid * NG, NG)], idx_v)
    gd = [None] * NG
    od = [None] * NG
    gd[0] = pltpu.async_copy(tab_hbm.at[idx_v.at[0]], bufs[0], gsem)
    for j in range(NG):
        if j + 1 < NG:
            if j - 1 >= 0:
                od[j - 1].wait()
            gd[j + 1] = pltpu.async_copy(tab_hbm.at[idx_v.at[j + 1]],
                                         bufs[(j + 1) % 2], gsem)
        gd[j].wait()
        od[j] = pltpu.async_copy(bufs[j % 2],
                                 emb_hbm.at[pl.ds(base + j * GI, GI)], osem)
    od[NG - 2].wait()
    od[NG - 1].wait()


TC_R = 2048


def _tc_lin_body(z1_ref, z2_ref, z3_ref, w1_ref, b1_ref, w2_ref, b2_ref,
                 w3_ref, b3_ref, o1_ref, o2_ref, o3_ref):
    z1 = z1_ref[...].reshape(TC_R, 1)
    z2 = z2_ref[...].reshape(TC_R, 1)
    z3 = z3_ref[...].reshape(TC_R, 1)
    o1_ref[...] = z1 * w1_ref[...].reshape(1, D1) + b1_ref[...].reshape(1, D1)
    o2_ref[...] = z2 * w2_ref[...].reshape(1, D) + b2_ref[...].reshape(1, D)
    o3_ref[...] = z3 * w3_ref[...].reshape(1, D) + b3_ref[...].reshape(1, D)


_tc_lin = pl.pallas_call(
    _tc_lin_body,
    grid=(B // TC_R,),
    in_specs=[
        pl.BlockSpec((TC_R,), lambda i: (i,)),
        pl.BlockSpec((TC_R,), lambda i: (i,)),
        pl.BlockSpec((TC_R,), lambda i: (i,)),
        pl.BlockSpec((D1,), lambda i: (0,)),
        pl.BlockSpec((D1,), lambda i: (0,)),
        pl.BlockSpec((D,), lambda i: (0,)),
        pl.BlockSpec((D,), lambda i: (0,)),
        pl.BlockSpec((D,), lambda i: (0,)),
        pl.BlockSpec((D,), lambda i: (0,)),
    ],
    out_specs=[
        pl.BlockSpec((TC_R, D1), lambda i: (i, 0)),
        pl.BlockSpec((TC_R, D), lambda i: (i, 0)),
        pl.BlockSpec((TC_R, D), lambda i: (i, 0)),
    ],
    out_shape=[
        jax.ShapeDtypeStruct((B, D1), jnp.float32),
        jax.ShapeDtypeStruct((B, D), jnp.float32),
        jax.ShapeDtypeStruct((B, D), jnp.float32),
    ],
)


def kernel(z, z1, z2, z3, emb_table, W1, b1, W2, b2, W3, b3):
    z2d = z.astype(jnp.int32).reshape(B // GI, GI)
    emb = _sc_emb(z2d, emb_table)
    o1, o2, o3 = _tc_lin(z1, z2, z3, W1.reshape(-1), b1,
                         W2.reshape(-1), b2, W3.reshape(-1), b3)
    return emb, o1, o2, o3


# speedup vs baseline: 1.8193x; 1.1424x over previous
import functools

import jax
import jax.numpy as jnp
from jax import lax
from jax.experimental import pallas as pl
from jax.experimental.pallas import tpu as pltpu
from jax.experimental.pallas import tpu_sc as plsc

NC = 2
NS = 16
L = 16
NW = NC * NS
B = 16384
D = 128
D1 = 50
BPW = B // NW
GI = 128
NG = BPW // GI

_mesh = plsc.VectorSubcoreMesh(core_axis_name="c", subcore_axis_name="s")


@functools.partial(
    pl.kernel,
    out_type=jax.ShapeDtypeStruct((B, D), jnp.float32),
    mesh=_mesh,
    compiler_params=pltpu.CompilerParams(needs_layout_passes=False),
    scratch_types=[
        pltpu.VMEM((NG, GI), jnp.int32),
        pltpu.VMEM((NG, GI, D), jnp.float32),
        [pltpu.SemaphoreType.DMA] * NG,
        pltpu.SemaphoreType.DMA,
    ],
)
def _sc_emb(z2d_hbm, tab_hbm, emb_hbm, idx_v, rows_v, gsems, osem):
    wid = lax.axis_index("s") * NC + lax.axis_index("c")
    base = wid * BPW

    pltpu.sync_copy(z2d_hbm.at[pl.ds(wid * NG, NG)], idx_v)
    gd = [pltpu.async_copy(tab_hbm.at[idx_v.at[j]], rows_v.at[j], gsems[j])
          for j in range(NG)]
    od = []
    for j in range(NG):
        gd[j].wait()
        od.append(pltpu.async_copy(rows_v.at[j],
                                   emb_hbm.at[pl.ds(base + j * GI, GI)], osem))
    for o in od:
        o.wait()


TC_R = 2048


def _tc_lin_body(z1_ref, z2_ref, z3_ref, w1_ref, b1_ref, w2_ref, b2_ref,
                 w3_ref, b3_ref, o1t_ref, o2_ref, o3_ref):
    z2 = z2_ref[...].reshape(TC_R, 1)
    z3 = z3_ref[...].reshape(TC_R, 1)
    z1 = z1_ref[...].reshape(1, TC_R)
    o1t_ref[...] = (w1_ref[...].reshape(D1, 1) * z1
                    + b1_ref[...].reshape(D1, 1))
    o2_ref[...] = z2 * w2_ref[...].reshape(1, D) + b2_ref[...].reshape(1, D)
    o3_ref[...] = z3 * w3_ref[...].reshape(1, D) + b3_ref[...].reshape(1, D)


_tc_lin = pl.pallas_call(
    _tc_lin_body,
    grid=(B // TC_R,),
    in_specs=[
        pl.BlockSpec((TC_R,), lambda i: (i,)),
        pl.BlockSpec((TC_R,), lambda i: (i,)),
        pl.BlockSpec((TC_R,), lambda i: (i,)),
        pl.BlockSpec((D1,), lambda i: (0,)),
        pl.BlockSpec((D1,), lambda i: (0,)),
        pl.BlockSpec((D,), lambda i: (0,)),
        pl.BlockSpec((D,), lambda i: (0,)),
        pl.BlockSpec((D,), lambda i: (0,)),
        pl.BlockSpec((D,), lambda i: (0,)),
    ],
    out_specs=[
        pl.BlockSpec((D1, TC_R), lambda i: (0, i)),
        pl.BlockSpec((TC_R, D), lambda i: (i, 0)),
        pl.BlockSpec((TC_R, D), lambda i: (i, 0)),
    ],
    out_shape=[
        jax.ShapeDtypeStruct((D1, B), jnp.float32),
        jax.ShapeDtypeStruct((B, D), jnp.float32),
        jax.ShapeDtypeStruct((B, D), jnp.float32),
    ],
)


def kernel(z, z1, z2, z3, emb_table, W1, b1, W2, b2, W3, b3):
    z2d = z.astype(jnp.int32).reshape(B // GI, GI)
    emb = _sc_emb(z2d, emb_table)
    o1t, o2, o3 = _tc_lin(z1, z2, z3, W1.reshape(-1), b1,
                          W2.reshape(-1), b2, W3.reshape(-1), b3)
    return emb, o1t.T, o2, o3
